# baseline (speedup 1.0000x reference)
import functools

import jax
import jax.numpy as jnp
from jax import lax
from jax.experimental import pallas as pl
from jax.experimental.pallas import tpu as pltpu
from jax.experimental.pallas import tpu_sc as plsc

NC = 2
NS = 16
NW = NC * NS


def _sc_gather(x, emb, nb):
  bsz, seq = x.shape
  d = emb.shape[1]
  g = 8
  nct = seq // g
  n = bsz * seq
  chunk = nb * g
  bpw = (bsz // nb) // NW
  npair = (nct + 1) // 2
  pairs_w = bpw * npair
  nhp = pairs_w // 2

  mesh = plsc.VectorSubcoreMesh(core_axis_name="c", subcore_axis_name="s")

  @functools.partial(
      pl.kernel,
      mesh=mesh,
      out_type=jax.ShapeDtypeStruct((n, d), jnp.int32),
      compiler_params=pltpu.CompilerParams(use_tc_tiling_on_sc=False),
      scratch_types=[
          pltpu.VMEM((nb, 2 * g), jnp.int32),
          pltpu.VMEM((nb, 2 * g), jnp.int32),
          pltpu.VMEM((chunk + 8,), jnp.int32),
          pltpu.VMEM((chunk + 16,), jnp.int32),
          pltpu.VMEM((chunk + 8,), jnp.int32),
          pltpu.VMEM((chunk + 16,), jnp.int32),
          pltpu.VMEM((chunk, d), jnp.int32),
          pltpu.VMEM((chunk, d), jnp.int32),
          pltpu.VMEM((chunk, d), jnp.int32),
          pltpu.VMEM((chunk, d), jnp.int32),
          pltpu.VMEM_SHARED(emb.shape, jnp.int32),
          pltpu.SemaphoreType.DMA,
          pltpu.SemaphoreType.DMA,
          pltpu.SemaphoreType.DMA,
          pltpu.SemaphoreType.DMA,
          pltpu.SemaphoreType.DMA,
          pltpu.SemaphoreType.DMA,
          pltpu.SemaphoreType.DMA,
          pltpu.SemaphoreType.DMA,
          pltpu.SemaphoreType.DMA,
          pltpu.SemaphoreType.DMA,
      ],
  )
  def k(x_hbm, emb_hbm, out_hbm,
        xb0, xb1, il0, ih0, il1, ih1, rl0, rh0, rl1, rh1, emb_v,
        xsem0, xsem1, gl0, gh0, gl1, gh1, ol0, oh0, ol1, oh1):
    wid = lax.axis_index("s") * NC + lax.axis_index("c")
    @pl.when(lax.axis_index("s") == 0)
    def _():
      pltpu.sync_copy(emb_hbm, emb_v)

    plsc.subcore_barrier()


    def coords(j):
      bb = j // npair
      p = j % npair
      b0 = (wid * bpw + bb) * nb
      p0 = jnp.minimum(p * 2 * g, seq - 2 * g)
      return b0, p0

    def xs(j, buf):
      b0, p0 = coords(j)
      return x_hbm.at[pl.ds(b0, nb), pl.ds(p0, 2 * g)]

    def outref(j, hi):
      b0, p0 = coords(j)
      ct = p0 // g + hi
      return out_hbm.at[pl.ds(ct * (bsz * g) + b0 * g, chunk)]

    def build(xb, il, ih):
      def body_lo(q, carry):
        il[pl.ds(q * g, 2 * g)] = xb[q]
        return carry
      lax.fori_loop(0, nb, body_lo, 0)

      def body_hi(i, carry):
        q = nb - 1 - i
        ih[pl.ds(q * g, 2 * g)] = xb[q]
        return carry
      lax.fori_loop(0, nb, body_hi, 0)

    def il_sl(il):
      return il.at[pl.ds(0, chunk)]

    def ih_sl(ih):
      return ih.at[pl.ds(g, chunk)]

    pltpu.async_copy(xs(0, 0), xb0, xsem0)
    pltpu.async_copy(xs(1, 1), xb1, xsem1)

    def halfstep(j, xb, il, ih, rl, rh, xsem, gl, gh, ol, oh, jj):
      pltpu.make_async_copy(xs(j, 0), xb, xsem).wait()
      build(xb, il, ih)

      @pl.when(jj < nhp - 1)
      def _():
        pltpu.async_copy(xs(j + 2, 0), xb, xsem)

      @pl.when(jj > 0)
      def _():
        pltpu.make_async_copy(rl, outref(j, 0), ol).wait()
        pltpu.make_async_copy(rh, outref(j, 1), oh).wait()

      pltpu.async_copy(emb_v.at[il_sl(il)], rl, gl)
      pltpu.async_copy(emb_v.at[ih_sl(ih)], rh, gh)

      pltpu.make_async_copy(emb_v.at[il_sl(il)], rl, gl).wait()
      pltpu.async_copy(rl, outref(j, 0), ol)
      pltpu.make_async_copy(emb_v.at[ih_sl(ih)], rh, gh).wait()
      pltpu.async_copy(rh, outref(j, 1), oh)

    def step(jj, carry):
      a = 2 * jj
      halfstep(a, xb0, il0, ih0, rl0, rh0, xsem0, gl0, gh0, ol0, oh0, jj)
      halfstep(a + 1, xb1, il1, ih1, rl1, rh1, xsem1, gl1, gh1, ol1, oh1, jj)
      return carry

    lax.fori_loop(0, nhp, step, 0)
    pltpu.make_async_copy(rl0, outref(pairs_w - 2, 0), ol0).wait()
    pltpu.make_async_copy(rh0, outref(pairs_w - 2, 1), oh0).wait()
    pltpu.make_async_copy(rl1, outref(pairs_w - 1, 0), ol1).wait()
    pltpu.make_async_copy(rh1, outref(pairs_w - 1, 1), oh1).wait()

  return k(x, emb)


def _mlp_body(e_ref, w1l_ref, w1h_ref, b1_ref, w2_ref, b2_ref, w3_ref,
              b3_ref, o_ref):
  nct, tb2, lw = e_ref.shape
  hw = lw // 2

  def mlp_tail(h):
    h = jnp.maximum(h + b1_ref[...], 0.0)
    h = jnp.dot(h, w2_ref[...], preferred_element_type=jnp.float32)
    h = jnp.maximum(h + b2_ref[...], 0.0)
    o = jnp.dot(h, w3_ref[...], preferred_element_type=jnp.float32)
    return o + b3_ref[...]

  hs = []
  for half in range(2):
    h = None
    for ct in range(nct):
      w = e_ref[ct][:, half * hw:(half + 1) * hw]
      flo = lax.bitcast_convert_type(w << 16, jnp.float32).astype(jnp.bfloat16)
      fhi = lax.bitcast_convert_type(
          w & jnp.int32(-65536), jnp.float32).astype(jnp.bfloat16)
      acc = jnp.dot(flo, w1l_ref[ct], preferred_element_type=jnp.float32)
      acc = acc + jnp.dot(fhi, w1h_ref[ct],
                          preferred_element_type=jnp.float32)
      h = acc if h is None else h + acc
    hs.append(mlp_tail(h))
  o = jnp.stack(hs, axis=1)
  o_ref[...] = o.reshape(2 * tb2, o.shape[-1])


def _tc_mlp(e4, W1l, W1h, b1, W2, b2, W3, b3, tb):
  nct, bsz2, lw = e4.shape
  bsz = 2 * bsz2
  h1 = W1l.shape[2]
  h2 = W2.shape[1]
  ncls = W3.shape[1]
  grid = (bsz // tb,)
  return pl.pallas_call(
      _mlp_body,
      grid=grid,
      in_specs=[
          pl.BlockSpec((nct, tb // 2, lw), lambda i: (0, i, 0)),
          pl.BlockSpec((nct, lw // 2, h1), lambda i: (0, 0, 0)),
          pl.BlockSpec((nct, lw // 2, h1), lambda i: (0, 0, 0)),
          pl.BlockSpec((1, h1), lambda i: (0, 0)),
          pl.BlockSpec((h1, h2), lambda i: (0, 0)),
          pl.BlockSpec((1, h2), lambda i: (0, 0)),
          pl.BlockSpec((h2, ncls), lambda i: (0, 0)),
          pl.BlockSpec((1, ncls), lambda i: (0, 0)),
      ],
      out_specs=pl.BlockSpec((tb, ncls), lambda i: (i, 0)),
      out_shape=jax.ShapeDtypeStruct((bsz, ncls), jnp.float32),
  )(e4, W1l, W1h, b1, W2, b2, W3, b3)


@jax.jit
def kernel(x, emb, W1, b1, W2, b2, W3, b3):
  b, s = x.shape
  v, d = emb.shape
  lw = 128
  nct = s * d // lw
  hd = d // 2
  perm = jnp.arange(d).reshape(2, hd).T.reshape(-1)
  emb_p = lax.bitcast_convert_type(
      emb[:, perm].astype(jnp.bfloat16).reshape(v, hd, 2), jnp.int32)
  rows = _sc_gather(x.astype(jnp.int32), emb_p, nb=128)
  e4 = rows.reshape(nct, b // 2, lw)
  w5 = W1.reshape(nct, lw // d, 2, hd, W1.shape[1])
  W1l = w5[:, :, 0].reshape(nct, lw // 2, W1.shape[1]).astype(jnp.bfloat16)
  W1h = w5[:, :, 1].reshape(nct, lw // 2, W1.shape[1]).astype(jnp.bfloat16)
  return _tc_mlp(e4, W1l, W1h, b1.reshape(1, -1), W2, b2.reshape(1, -1),
                 W3, b3.reshape(1, -1), tb=512)


# speedup vs baseline: 1.1345x; 1.1345x over previous
import functools

import jax
import jax.numpy as jnp
from jax import lax
from jax.experimental import pallas as pl
from jax.experimental.pallas import tpu as pltpu
from jax.experimental.pallas import tpu_sc as plsc

NC = 2
NS = 16
NW = NC * NS


def _sc_gather(x, emb, nb):
  bsz, seq = x.shape
  d = emb.shape[1]
  g = 128 // d
  nct = seq // g
  n = bsz * seq
  chunk = nb * g
  bpw = (bsz // nb) // NW
  npair = (nct + 1) // 2
  pairs_w = bpw * npair
  nhp = pairs_w // 2

  mesh = plsc.VectorSubcoreMesh(core_axis_name="c", subcore_axis_name="s")

  @functools.partial(
      pl.kernel,
      mesh=mesh,
      out_type=jax.ShapeDtypeStruct((n, d), jnp.float32),
      compiler_params=pltpu.CompilerParams(use_tc_tiling_on_sc=False),
      scratch_types=[
          pltpu.VMEM((nb, 2 * g), jnp.int32),
          pltpu.VMEM((nb, 2 * g), jnp.int32),
          pltpu.VMEM((chunk + 8,), jnp.int32),
          pltpu.VMEM((chunk + 16,), jnp.int32),
          pltpu.VMEM((chunk + 8,), jnp.int32),
          pltpu.VMEM((chunk + 16,), jnp.int32),
          pltpu.VMEM((chunk, d), jnp.float32),
          pltpu.VMEM((chunk, d), jnp.float32),
          pltpu.VMEM((chunk, d), jnp.float32),
          pltpu.VMEM((chunk, d), jnp.float32),
          pltpu.VMEM_SHARED(emb.shape, jnp.float32),
          pltpu.SemaphoreType.DMA,
          pltpu.SemaphoreType.DMA,
          pltpu.SemaphoreType.DMA,
          pltpu.SemaphoreType.DMA,
          pltpu.SemaphoreType.DMA,
          pltpu.SemaphoreType.DMA,
          pltpu.SemaphoreType.DMA,
          pltpu.SemaphoreType.DMA,
          pltpu.SemaphoreType.DMA,
          pltpu.SemaphoreType.DMA,
      ],
  )
  def k(x_hbm, emb_hbm, out_hbm,
        xb0, xb1, il0, ih0, il1, ih1, rl0, rh0, rl1, rh1, emb_v,
        xsem0, xsem1, gl0, gh0, gl1, gh1, ol0, oh0, ol1, oh1):
    wid = lax.axis_index("s") * NC + lax.axis_index("c")
    @pl.when(lax.axis_index("s") == 0)
    def _():
      pltpu.sync_copy(emb_hbm, emb_v)

    plsc.subcore_barrier()


    def coords(j):
      bb = j // npair
      p = j % npair
      b0 = (wid * bpw + bb) * nb
      p0 = jnp.minimum(p * 2 * g, seq - 2 * g)
      return b0, p0

    def xs(j, buf):
      b0, p0 = coords(j)
      return x_hbm.at[pl.ds(b0, nb), pl.ds(p0, 2 * g)]

    def outref(j, hi):
      b0, p0 = coords(j)
      ct = p0 // g + hi
      return out_hbm.at[pl.ds(ct * (bsz * g) + b0 * g, chunk)]

    def build(xb, il, ih):
      def body_lo(q, carry):
        il[pl.ds(q * g, 2 * g)] = xb[q]
        return carry
      lax.fori_loop(0, nb, body_lo, 0)

      def body_hi(i, carry):
        q = nb - 1 - i
        ih[pl.ds(q * g, 2 * g)] = xb[q]
        return carry
      lax.fori_loop(0, nb, body_hi, 0)

    def il_sl(il):
      return il.at[pl.ds(0, chunk)]

    def ih_sl(ih):
      return ih.at[pl.ds(g, chunk)]

    pltpu.async_copy(xs(0, 0), xb0, xsem0)
    pltpu.async_copy(xs(1, 1), xb1, xsem1)

    def halfstep(j, xb, il, ih, rl, rh, xsem, gl, gh, ol, oh, jj):
      pltpu.make_async_copy(xs(j, 0), xb, xsem).wait()
      build(xb, il, ih)

      @pl.when(jj < nhp - 1)
      def _():
        pltpu.async_copy(xs(j + 2, 0), xb, xsem)

      @pl.when(jj > 0)
      def _():
        pltpu.make_async_copy(rl, outref(j, 0), ol).wait()
        pltpu.make_async_copy(rh, outref(j, 1), oh).wait()

      pltpu.async_copy(emb_v.at[il_sl(il)], rl, gl)
      pltpu.async_copy(emb_v.at[ih_sl(ih)], rh, gh)

      pltpu.make_async_copy(emb_v.at[il_sl(il)], rl, gl).wait()
      pltpu.async_copy(rl, outref(j, 0), ol)
      pltpu.make_async_copy(emb_v.at[ih_sl(ih)], rh, gh).wait()
      pltpu.async_copy(rh, outref(j, 1), oh)

    def step(jj, carry):
      a = 2 * jj
      halfstep(a, xb0, il0, ih0, rl0, rh0, xsem0, gl0, gh0, ol0, oh0, jj)
      halfstep(a + 1, xb1, il1, ih1, rl1, rh1, xsem1, gl1, gh1, ol1, oh1, jj)
      return carry

    lax.fori_loop(0, nhp, step, 0)
    pltpu.make_async_copy(rl0, outref(pairs_w - 2, 0), ol0).wait()
    pltpu.make_async_copy(rh0, outref(pairs_w - 2, 1), oh0).wait()
    pltpu.make_async_copy(rl1, outref(pairs_w - 1, 0), ol1).wait()
    pltpu.make_async_copy(rh1, outref(pairs_w - 1, 1), oh1).wait()

  return k(x, emb)


def _mlp_body(e_ref, w1_ref, b1_ref, w2_ref, b2_ref, w3_ref, b3_ref, o_ref):
  nct = e_ref.shape[0]
  h = jnp.dot(e_ref[0], w1_ref[0], preferred_element_type=jnp.float32)
  for ct in range(1, nct):
    h = h + jnp.dot(e_ref[ct], w1_ref[ct], preferred_element_type=jnp.float32)
  h = jnp.maximum(h + b1_ref[...], 0.0)
  h = jnp.dot(h, w2_ref[...], preferred_element_type=jnp.float32)
  h = jnp.maximum(h + b2_ref[...], 0.0)
  o = jnp.dot(h, w3_ref[...], preferred_element_type=jnp.float32)
  o_ref[...] = o + b3_ref[...]


def _tc_mlp(e3, W1r, b1, W2, b2, W3, b3, tb):
  nct, bsz, lw = e3.shape
  h1 = W1r.shape[2]
  h2 = W2.shape[1]
  ncls = W3.shape[1]
  grid = (bsz // tb,)
  return pl.pallas_call(
      _mlp_body,
      grid=grid,
      in_specs=[
          pl.BlockSpec((nct, tb, lw), lambda i: (0, i, 0)),
          pl.BlockSpec((nct, lw, h1), lambda i: (0, 0, 0)),
          pl.BlockSpec((1, h1), lambda i: (0, 0)),
          pl.BlockSpec((h1, h2), lambda i: (0, 0)),
          pl.BlockSpec((1, h2), lambda i: (0, 0)),
          pl.BlockSpec((h2, ncls), lambda i: (0, 0)),
          pl.BlockSpec((1, ncls), lambda i: (0, 0)),
      ],
      out_specs=pl.BlockSpec((tb, ncls), lambda i: (i, 0)),
      out_shape=jax.ShapeDtypeStruct((bsz, ncls), jnp.float32),
  )(e3, W1r, b1, W2, b2, W3, b3)


@jax.jit
def kernel(x, emb, W1, b1, W2, b2, W3, b3):
  b, s = x.shape
  v, d = emb.shape
  lw = 128
  nct = s * d // lw
  rows = _sc_gather(x.astype(jnp.int32), emb, nb=128)
  e3 = rows.reshape(nct, b, lw)
  W1r = W1.reshape(nct, lw, W1.shape[1])
  return _tc_mlp(e3, W1r, b1.reshape(1, -1), W2, b2.reshape(1, -1),
                 W3, b3.reshape(1, -1), tb=1024)


# speedup vs baseline: 1.1759x; 1.0365x over previous
import functools

import jax
import jax.numpy as jnp
from jax import lax
from jax.experimental import pallas as pl
from jax.experimental.pallas import tpu as pltpu
from jax.experimental.pallas import tpu_sc as plsc

NC = 2
NS = 16
NW = NC * NS
PB = 16


def _sc_pair_gather(x, ptab, v, nb):
  bsz, seq = x.shape
  w = ptab.shape[1]
  nblk = (seq + PB - 1) // PB
  chunk = nb * (PB // 2)
  n = nblk * bsz * (PB // 2)
  bpw = (bsz // nb) // NW
  iters_w = bpw * nblk
  nh = iters_w // 2

  mesh = plsc.VectorSubcoreMesh(core_axis_name="c", subcore_axis_name="s")

  @functools.partial(
      pl.kernel,
      mesh=mesh,
      out_type=jax.ShapeDtypeStruct((n, w), jnp.int32),
      compiler_params=pltpu.CompilerParams(use_tc_tiling_on_sc=False),
      scratch_types=[
          pltpu.VMEM((nb, PB), jnp.int32),
          pltpu.VMEM((nb, PB), jnp.int32),
          pltpu.VMEM((nb * PB + 8,), jnp.int32),
          pltpu.VMEM((nb * PB + 8,), jnp.int32),
          pltpu.VMEM((chunk + 8,), jnp.int32),
          pltpu.VMEM((chunk + 8,), jnp.int32),
          pltpu.VMEM((chunk, w), jnp.int32),
          pltpu.VMEM((chunk, w), jnp.int32),
          pltpu.VMEM_SHARED(ptab.shape, jnp.int32),
          pltpu.SemaphoreType.DMA,
          pltpu.SemaphoreType.DMA,
          pltpu.SemaphoreType.DMA,
          pltpu.SemaphoreType.DMA,
          pltpu.SemaphoreType.DMA,
          pltpu.SemaphoreType.DMA,
      ],
  )
  def k(x_hbm, ptab_hbm, out_hbm,
        xb0, xb1, xf0, xf1, px0, px1, r0, r1, tab,
        xsem0, xsem1, gsem0, gsem1, osem0, osem1):
    wid = lax.axis_index("s") * NC + lax.axis_index("c")
    @pl.when(lax.axis_index("s") == 0)
    def _():
      pltpu.sync_copy(ptab_hbm, tab)

    plsc.subcore_barrier()

    def coords(j):
      bb = j // nblk
      blk = j % nblk
      b0 = (wid * bpw + bb) * nb
      p0 = jnp.minimum(blk * PB, seq - PB)
      return b0, blk, p0

    def xs(j):
      b0, blk, p0 = coords(j)
      return x_hbm.at[pl.ds(b0, nb), pl.ds(p0, PB)]

    def outref(j):
      b0, blk, p0 = coords(j)
      return out_hbm.at[pl.ds(blk * (bsz * (PB // 2)) + b0 * (PB // 2),
                              chunk)]

    def build(xb, xf, px):
      def flat(q, carry):
        xf[pl.ds(q * PB, PB)] = xb[q]
        return carry
      lax.fori_loop(0, nb, flat, 0)

      def pairs(q, carry):
        a = xf[pl.ds(q * PB, PB)]
        b = xf[pl.ds(q * PB + PB // 2, PB)]
        px[pl.ds(q * (PB // 2), PB)] = a * v + b
        return carry
      lax.fori_loop(0, nb, pairs, 0)

    def px_sl(px):
      return px.at[pl.ds(0, chunk)]

    pltpu.async_copy(xs(0), xb0, xsem0)
    pltpu.async_copy(xs(1), xb1, xsem1)

    def halfstep(j, xb, xf, px, rows, xsem, gsem, osem, jj):
      pltpu.make_async_copy(xs(j), xb, xsem).wait()
      build(xb, xf, px)

      @pl.when(jj < nh - 1)
      def _():
        pltpu.async_copy(xs(j + 2), xb, xsem)

      @pl.when(jj > 0)
      def _():
        pltpu.make_async_copy(rows, outref(j), osem).wait()

      pltpu.async_copy(tab.at[px_sl(px)], rows, gsem)
      pltpu.make_async_copy(tab.at[px_sl(px)], rows, gsem).wait()
      pltpu.async_copy(rows, outref(j), osem)

    def step(jj, carry):
      a = 2 * jj
      halfstep(a, xb0, xf0, px0, r0, xsem0, gsem0, osem0, jj)
      halfstep(a + 1, xb1, xf1, px1, r1, xsem1, gsem1, osem1, jj)
      return carry

    lax.fori_loop(0, nh, step, 0)
    pltpu.make_async_copy(r0, outref(iters_w - 2), osem0).wait()
    pltpu.make_async_copy(r1, outref(iters_w - 1), osem1).wait()

  return k(x, ptab)


def _mlp_body(e_ref, w1l_ref, w1h_ref, b1_ref, w2_ref, b2_ref, w3_ref,
              b3_ref, o_ref):
  nblk = e_ref.shape[0]
  h = None
  for blk in range(nblk):
    wds = e_ref[blk]
    flo = lax.bitcast_convert_type(wds << 16, jnp.float32)
    flo = flo.astype(jnp.bfloat16)
    fhi = lax.bitcast_convert_type(wds & jnp.int32(-65536), jnp.float32)
    fhi = fhi.astype(jnp.bfloat16)
    acc = jnp.dot(flo, w1l_ref[blk], preferred_element_type=jnp.float32)
    acc = acc + jnp.dot(fhi, w1h_ref[blk], preferred_element_type=jnp.float32)
    h = acc if h is None else h + acc
  h = jnp.maximum(h + b1_ref[...], 0.0)
  h = jnp.dot(h, w2_ref[...], preferred_element_type=jnp.float32)
  h = jnp.maximum(h + b2_ref[...], 0.0)
  o = jnp.dot(h, w3_ref[...], preferred_element_type=jnp.float32)
  o_ref[...] = o + b3_ref[...]


def _tc_mlp(e5, W1l, W1h, b1, W2, b2, W3, b3, tb):
  nblk, bsz, lw = e5.shape
  h1 = W1l.shape[2]
  h2 = W2.shape[1]
  ncls = W3.shape[1]
  grid = (bsz // tb,)
  return pl.pallas_call(
      _mlp_body,
      grid=grid,
      in_specs=[
          pl.BlockSpec((nblk, tb, lw), lambda i: (0, i, 0)),
          pl.BlockSpec((nblk, lw, h1), lambda i: (0, 0, 0)),
          pl.BlockSpec((nblk, lw, h1), lambda i: (0, 0, 0)),
          pl.BlockSpec((1, h1), lambda i: (0, 0)),
          pl.BlockSpec((h1, h2), lambda i: (0, 0)),
          pl.BlockSpec((1, h2), lambda i: (0, 0)),
          pl.BlockSpec((h2, ncls), lambda i: (0, 0)),
          pl.BlockSpec((1, ncls), lambda i: (0, 0)),
      ],
      out_specs=pl.BlockSpec((tb, ncls), lambda i: (i, 0)),
      out_shape=jax.ShapeDtypeStruct((bsz, ncls), jnp.float32),
  )(e5, W1l, W1h, b1, W2, b2, W3, b3)


@jax.jit
def kernel(x, emb, W1, b1, W2, b2, W3, b3):
  b, s = x.shape
  v, d = emb.shape
  hd = d // 2
  nblk = (s + PB - 1) // PB
  h1 = W1.shape[1]

  perm = jnp.arange(d).reshape(2, hd).T.reshape(-1)
  embi = emb[:, perm].astype(jnp.bfloat16)
  ptab_bf = jnp.concatenate(
      [jnp.repeat(embi, v, axis=0), jnp.tile(embi, (v, 1))], axis=1)
  ptab = lax.bitcast_convert_type(
      ptab_bf.reshape(v * v, d, 2), jnp.int32)

  rows = _sc_pair_gather(x.astype(jnp.int32), ptab, v, nb=128)
  e5 = rows.reshape(nblk, b, 128)

  lane = jnp.arange(128)
  k = lane // 16
  hh = (lane % 16) // 8
  j = lane % 8
  p0s = jnp.minimum(jnp.arange(nblk) * PB, s - PB)
  pos = p0s[:, None] + k[None, :] + 8 * hh[None, :]
  w3d = W1.reshape(s, d, h1)
  jb = jnp.broadcast_to(j[None, :], pos.shape)
  W1l = w3d[pos, jb]
  W1h = w3d[pos, jb + hd]
  dup = (jnp.arange(nblk)[:, None] * PB > pos)
  W1l = jnp.where(dup[..., None], 0.0, W1l).astype(jnp.bfloat16)
  W1h = jnp.where(dup[..., None], 0.0, W1h).astype(jnp.bfloat16)

  return _tc_mlp(e5, W1l, W1h, b1.reshape(1, -1), W2, b2.reshape(1, -1),
                 W3, b3.reshape(1, -1), tb=1024)


# speedup vs baseline: 1.1831x; 1.0061x over previous
import functools

import jax
import jax.numpy as jnp
from jax import lax
from jax.experimental import pallas as pl
from jax.experimental.pallas import tpu as pltpu
from jax.experimental.pallas import tpu_sc as plsc

NC = 2
NS = 16
NW = NC * NS
PB = 16


def _sc_pair_gather(x, ptab, v, nb):
  bsz, seq = x.shape
  w = ptab.shape[1]
  nblk = (seq + PB - 1) // PB
  chunk = nb * (PB // 2)
  n = nblk * bsz * (PB // 2)
  bpw = (bsz // nb) // NW
  iters_w = bpw * nblk
  nh = iters_w // 2

  mesh = plsc.VectorSubcoreMesh(core_axis_name="c", subcore_axis_name="s")

  @functools.partial(
      pl.kernel,
      mesh=mesh,
      out_type=jax.ShapeDtypeStruct((n, w), jnp.int32),
      compiler_params=pltpu.CompilerParams(use_tc_tiling_on_sc=False),
      scratch_types=[
          pltpu.VMEM((nb, PB), jnp.int32),
          pltpu.VMEM((nb, PB), jnp.int32),
          pltpu.VMEM((nb * PB + 8,), jnp.int32),
          pltpu.VMEM((nb * PB + 8,), jnp.int32),
          pltpu.VMEM((chunk + 8,), jnp.int32),
          pltpu.VMEM((chunk + 8,), jnp.int32),
          pltpu.VMEM((chunk, w), jnp.int32),
          pltpu.VMEM((chunk, w), jnp.int32),
          pltpu.VMEM_SHARED(ptab.shape, jnp.int32),
          pltpu.SemaphoreType.DMA,
          pltpu.SemaphoreType.DMA,
          pltpu.SemaphoreType.DMA,
          pltpu.SemaphoreType.DMA,
          pltpu.SemaphoreType.DMA,
          pltpu.SemaphoreType.DMA,
      ],
  )
  def k(x_hbm, ptab_hbm, out_hbm,
        xb0, xb1, xf0, xf1, px0, px1, r0, r1, tab,
        xsem0, xsem1, gsem0, gsem1, osem0, osem1):
    wid = lax.axis_index("s") * NC + lax.axis_index("c")
    @pl.when(lax.axis_index("s") == 0)
    def _():
      pltpu.sync_copy(ptab_hbm, tab)

    plsc.subcore_barrier()

    def coords(j):
      bb = j // nblk
      blk = j % nblk
      b0 = (wid * bpw + bb) * nb
      p0 = jnp.minimum(blk * PB, seq - PB)
      return b0, blk, p0

    def xs(j):
      b0, blk, p0 = coords(j)
      return x_hbm.at[pl.ds(b0, nb), pl.ds(p0, PB)]

    def outref(j):
      b0, blk, p0 = coords(j)
      return out_hbm.at[pl.ds(blk * (bsz * (PB // 2)) + b0 * (PB // 2),
                              chunk)]

    def build(xb, xf, px):
      def flat(q, carry):
        xf[pl.ds(q * PB, PB)] = xb[q]
        return carry
      lax.fori_loop(0, nb, flat, 0)

      def pairs(q, carry):
        a = xf[pl.ds(q * PB, PB)]
        b = xf[pl.ds(q * PB + PB // 2, PB)]
        px[pl.ds(q * (PB // 2), PB)] = a * v + b
        return carry
      lax.fori_loop(0, nb, pairs, 0)

    def px_sl(px):
      return px.at[pl.ds(0, chunk)]

    pltpu.async_copy(xs(0), xb0, xsem0)
    pltpu.async_copy(xs(1), xb1, xsem1)

    def halfstep(j, xb, xf, px, rows, xsem, gsem, osem, jj):
      pltpu.make_async_copy(xs(j), xb, xsem).wait()
      build(xb, xf, px)

      @pl.when(jj < nh - 1)
      def _():
        pltpu.async_copy(xs(j + 2), xb, xsem)

      @pl.when(jj > 0)
      def _():
        pltpu.make_async_copy(rows, outref(j), osem).wait()

      pltpu.async_copy(tab.at[px_sl(px)], rows, gsem)
      pltpu.make_async_copy(tab.at[px_sl(px)], rows, gsem).wait()
      pltpu.async_copy(rows, outref(j), osem)

    def step(jj, carry):
      a = 2 * jj
      halfstep(a, xb0, xf0, px0, r0, xsem0, gsem0, osem0, jj)
      halfstep(a + 1, xb1, xf1, px1, r1, xsem1, gsem1, osem1, jj)
      return carry

    lax.fori_loop(0, nh, step, 0)
    pltpu.make_async_copy(r0, outref(iters_w - 2), osem0).wait()
    pltpu.make_async_copy(r1, outref(iters_w - 1), osem1).wait()

  return k(x, ptab)


def _mlp_body(e_ref, w1l_ref, w1h_ref, b1_ref, w2_ref, b2_ref, w3_ref,
              b3_ref, o_ref):
  nblk = e_ref.shape[0]
  h = None
  for blk in range(nblk):
    wds = e_ref[blk]
    flo = lax.bitcast_convert_type(wds << 16, jnp.float32)
    flo = flo.astype(jnp.bfloat16)
    fhi = lax.bitcast_convert_type(wds & jnp.int32(-65536), jnp.float32)
    fhi = fhi.astype(jnp.bfloat16)
    acc = jnp.dot(flo, w1l_ref[blk], preferred_element_type=jnp.float32)
    acc = acc + jnp.dot(fhi, w1h_ref[blk], preferred_element_type=jnp.float32)
    h = acc if h is None else h + acc
  h = jnp.maximum(h + b1_ref[...], 0.0)
  h = jnp.dot(h, w2_ref[...], preferred_element_type=jnp.float32)
  h = jnp.maximum(h + b2_ref[...], 0.0)
  o = jnp.dot(h, w3_ref[...], preferred_element_type=jnp.float32)
  o_ref[...] = o + b3_ref[...]


def _tc_mlp(e5, W1l, W1h, b1, W2, b2, W3, b3, tb):
  nblk, bsz, lw = e5.shape
  h1 = W1l.shape[2]
  h2 = W2.shape[1]
  ncls = W3.shape[1]
  grid = (bsz // tb,)
  return pl.pallas_call(
      _mlp_body,
      grid=grid,
      in_specs=[
          pl.BlockSpec((nblk, tb, lw), lambda i: (0, i, 0)),
          pl.BlockSpec((nblk, lw, h1), lambda i: (0, 0, 0)),
          pl.BlockSpec((nblk, lw, h1), lambda i: (0, 0, 0)),
          pl.BlockSpec((1, h1), lambda i: (0, 0)),
          pl.BlockSpec((h1, h2), lambda i: (0, 0)),
          pl.BlockSpec((1, h2), lambda i: (0, 0)),
          pl.BlockSpec((h2, ncls), lambda i: (0, 0)),
          pl.BlockSpec((1, ncls), lambda i: (0, 0)),
      ],
      out_specs=pl.BlockSpec((tb, ncls), lambda i: (i, 0)),
      out_shape=jax.ShapeDtypeStruct((bsz, ncls), jnp.float32),
  )(e5, W1l, W1h, b1, W2, b2, W3, b3)


@jax.jit
def kernel(x, emb, W1, b1, W2, b2, W3, b3):
  b, s = x.shape
  v, d = emb.shape
  hd = d // 2
  nblk = (s + PB - 1) // PB
  h1 = W1.shape[1]

  perm = jnp.arange(d).reshape(2, hd).T.reshape(-1)
  embi = emb[:, perm].astype(jnp.bfloat16)
  ptab_bf = jnp.concatenate(
      [jnp.repeat(embi, v, axis=0), jnp.tile(embi, (v, 1))], axis=1)
  ptab = lax.bitcast_convert_type(
      ptab_bf.reshape(v * v, d, 2), jnp.int32)

  rows = _sc_pair_gather(x.astype(jnp.int32), ptab, v, nb=256)
  e5 = rows.reshape(nblk, b, 128)

  lane = jnp.arange(128)
  k = lane // 16
  hh = (lane % 16) // 8
  j = lane % 8
  p0s = jnp.minimum(jnp.arange(nblk) * PB, s - PB)
  pos = p0s[:, None] + k[None, :] + 8 * hh[None, :]
  w3d = W1.reshape(s, d, h1)
  jb = jnp.broadcast_to(j[None, :], pos.shape)
  W1l = w3d[pos, jb]
  W1h = w3d[pos, jb + hd]
  dup = (jnp.arange(nblk)[:, None] * PB > pos)
  W1l = jnp.where(dup[..., None], 0.0, W1l).astype(jnp.bfloat16)
  W1h = jnp.where(dup[..., None], 0.0, W1h).astype(jnp.bfloat16)

  return _tc_mlp(e5, W1l, W1h, b1.reshape(1, -1), W2, b2.reshape(1, -1),
                 W3, b3.reshape(1, -1), tb=1024)
